# baseline (speedup 1.0000x reference)
import jax
import jax.numpy as jnp
from jax.experimental import pallas as pl
from jax.experimental.pallas import tpu as pltpu

_NUM_EXPERTS = 16
_INV_TEMP = 2.0
_BB = 8


def _router_body(x_ref, wt_ref, b_ref, o_ref):
    pooled = jnp.mean(x_ref[...], axis=2)
    logits = jnp.dot(pooled, wt_ref[...],
                     preferred_element_type=jnp.float32)
    logits = (logits + b_ref[...]) * _INV_TEMP
    m = jnp.max(logits, axis=-1, keepdims=True)
    e = jnp.exp(logits - m)
    o_ref[...] = e / jnp.sum(e, axis=-1, keepdims=True)


def kernel(x, W, b):
    B, C = x.shape[0], x.shape[1]
    HW = 1
    for d in x.shape[2:]:
        HW *= d
    x3 = x.reshape(B, C, HW)
    wt = W.T
    b2 = b.reshape(1, _NUM_EXPERTS)
    return pl.pallas_call(
        _router_body,
        grid=(B // _BB,),
        in_specs=[
            pl.BlockSpec((_BB, C, HW), lambda i: (i, 0, 0)),
            pl.BlockSpec((C, _NUM_EXPERTS), lambda i: (0, 0)),
            pl.BlockSpec((1, _NUM_EXPERTS), lambda i: (0, 0)),
        ],
        out_specs=pl.BlockSpec((_BB, _NUM_EXPERTS), lambda i: (i, 0)),
        out_shape=jax.ShapeDtypeStruct((B, _NUM_EXPERTS), jnp.float32),
    )(x3, wt, b2)
